# Initial kernel scaffold; baseline (speedup 1.0000x reference)
import jax
import jax.numpy as jnp
from jax.experimental import pallas as pl


def kernel(logits, labels):
    raise NotImplementedError("write your pallas kernel here")


# speedup vs baseline: 5.6467x; 5.6467x over previous
import jax
import jax.numpy as jnp
from jax.experimental import pallas as pl
from jax.experimental.pallas import tpu as pltpu

_C = 19
_CH = 16384
_KEEP_RATIO = 0.25

_INT_MIN = -2147483648
_POS_MASK = 0x7FFFFFFF


def _monotonic_key(x):
    i = jax.lax.bitcast_convert_type(x, jnp.int32)
    return jnp.where(i >= 0, i, i ^ _POS_MASK)


def _key_to_float(m):
    return jax.lax.bitcast_convert_type(
        jnp.where(m >= 0, m, m ^ _POS_MASK), jnp.float32)


def _ohem_kernel(logits_ref, labels_ref, out_ref, keys_scratch, *, n_steps, k):
    s = pl.program_id(0)
    x = logits_ref[0]
    lab = labels_ref[0]

    mx = jnp.max(x, axis=0, keepdims=True)
    lse = mx + jnp.log(jnp.sum(jnp.exp(x - mx), axis=0, keepdims=True))
    cls = jax.lax.broadcasted_iota(jnp.int32, x.shape, 0)
    sel = jnp.sum(jnp.where(cls == lab, x, 0.0), axis=0, keepdims=True)
    loss = lse - sel
    keys_scratch[pl.ds(s, 1), :] = _monotonic_key(loss)

    @pl.when(s == n_steps - 1)
    def _select():
        def body(it, t_u):
            cand = t_u | (jnp.int32(1) << (31 - it))
            thr = cand ^ _INT_MIN
            cnt = jnp.sum((keys_scratch[...] >= thr).astype(jnp.int32))
            return jnp.where(cnt >= k, cand, t_u)

        t_u = jax.lax.fori_loop(0, 32, body, jnp.int32(0))
        t_m = t_u ^ _INT_MIN

        data = keys_scratch[...]
        gt = data > t_m
        cnt_gt = jnp.sum(gt.astype(jnp.int32))
        sum_gt = jnp.sum(jnp.where(gt, _key_to_float(data), 0.0))
        t_f = _key_to_float(t_m)
        out_ref[0, 0] = (sum_gt + (k - cnt_gt).astype(jnp.float32) * t_f) \
            / jnp.float32(k)


def kernel(logits, labels):
    b, c, h, w = logits.shape
    hw = h * w
    n_chunks = hw // _CH
    n_steps = b * n_chunks
    k = max(1, int(b * hw * _KEEP_RATIO))

    logits3 = logits.reshape(b, c, hw)
    labels3 = labels.reshape(b, 1, hw)

    import functools
    body = functools.partial(_ohem_kernel, n_steps=n_steps, k=k)

    out = pl.pallas_call(
        body,
        grid=(n_steps,),
        in_specs=[
            pl.BlockSpec((1, c, _CH), lambda s: (s // n_chunks, 0, s % n_chunks)),
            pl.BlockSpec((1, 1, _CH), lambda s: (s // n_chunks, 0, s % n_chunks)),
        ],
        out_specs=pl.BlockSpec(memory_space=pltpu.SMEM),
        out_shape=jax.ShapeDtypeStruct((1, 1), jnp.float32),
        scratch_shapes=[pltpu.VMEM((n_steps, _CH), jnp.int32)],
    )(logits3, labels3)
    return out[0, 0]


# speedup vs baseline: 7.7923x; 1.3800x over previous
import functools

import jax
import jax.numpy as jnp
from jax.experimental import pallas as pl
from jax.experimental.pallas import tpu as pltpu

_C = 19
_SUB = 16
_LANE = 4096
_KEEP_RATIO = 0.25

_INT_MIN = -2147483648
_POS_MASK = 0x7FFFFFFF


def _monotonic_key(x):
    i = jax.lax.bitcast_convert_type(x, jnp.int32)
    return jnp.where(i >= 0, i, i ^ _POS_MASK)


def _key_to_float(m):
    return jax.lax.bitcast_convert_type(
        jnp.where(m >= 0, m, m ^ _POS_MASK), jnp.float32)


def _ohem_kernel(logits_ref, labels_ref, out_ref, keys_scratch, *, n_steps, k):
    s = pl.program_id(0)
    x = logits_ref[0]
    lab = labels_ref[0, 0]

    mx = jnp.max(x, axis=0)
    lse = mx + jnp.log(jnp.sum(jnp.exp(x - mx[None]), axis=0))
    cls = jax.lax.broadcasted_iota(jnp.int32, x.shape, 0)
    sel = jnp.sum(jnp.where(cls == lab[None], x, 0.0), axis=0)
    loss = lse - sel
    keys_scratch[pl.ds(s * _SUB, _SUB), :] = _monotonic_key(loss)

    @pl.when(s == n_steps - 1)
    def _select():
        def body(it, t_u):
            cand = t_u | (jnp.int32(1) << (31 - it))
            thr = cand ^ _INT_MIN
            cnt = jnp.sum((keys_scratch[...] >= thr).astype(jnp.int32))
            return jnp.where(cnt >= k, cand, t_u)

        t_u = jax.lax.fori_loop(0, 32, body, jnp.int32(0))
        t_m = t_u ^ _INT_MIN

        data = keys_scratch[...]
        gt = data > t_m
        cnt_gt = jnp.sum(gt.astype(jnp.int32))
        sum_gt = jnp.sum(jnp.where(gt, _key_to_float(data), 0.0))
        t_f = _key_to_float(t_m)
        out_ref[0, 0] = (sum_gt + (k - cnt_gt).astype(jnp.float32) * t_f) \
            / jnp.float32(k)


def kernel(logits, labels):
    b, c, h, w = logits.shape
    hw = h * w
    ch = _SUB * _LANE
    rows = hw // _LANE
    n_chunks = rows // _SUB
    n_steps = b * n_chunks
    k = max(1, int(b * hw * _KEEP_RATIO))

    logits4 = logits.reshape(b, c, rows, _LANE)
    labels4 = labels.reshape(b, 1, rows, _LANE)

    body = functools.partial(_ohem_kernel, n_steps=n_steps, k=k)

    out = pl.pallas_call(
        body,
        grid=(n_steps,),
        in_specs=[
            pl.BlockSpec((1, c, _SUB, _LANE),
                         lambda s: (s // n_chunks, 0, s % n_chunks, 0)),
            pl.BlockSpec((1, 1, _SUB, _LANE),
                         lambda s: (s // n_chunks, 0, s % n_chunks, 0)),
        ],
        out_specs=pl.BlockSpec(memory_space=pltpu.SMEM),
        out_shape=jax.ShapeDtypeStruct((1, 1), jnp.float32),
        scratch_shapes=[pltpu.VMEM((n_steps * _SUB, _LANE), jnp.int32)],
    )(logits4, labels4)
    return out[0, 0]


# speedup vs baseline: 8.8903x; 1.1409x over previous
import functools

import jax
import jax.numpy as jnp
from jax.experimental import pallas as pl
from jax.experimental.pallas import tpu as pltpu

_C = 19
_SUB = 16
_LANE = 4096
_KEEP_RATIO = 0.25


def _ohem_kernel(logits_ref, labels_ref, out_ref, keys_scratch, hi_scratch,
                 lo_scratch, *, n_steps, k):
    s = pl.program_id(0)
    x = logits_ref[0]
    lab = labels_ref[0, 0]

    mx = jnp.max(x, axis=0)
    lse = mx + jnp.log(jnp.sum(jnp.exp(x - mx[None]), axis=0))
    cls = jax.lax.broadcasted_iota(jnp.int32, x.shape, 0)
    sel = jnp.sum(jnp.where(cls == lab[None], x, 0.0), axis=0)
    loss = lse - sel
    key = jax.lax.bitcast_convert_type(loss, jnp.int32)
    rows = pl.ds(s * _SUB, _SUB)
    keys_scratch[rows, :] = key
    hi_scratch[rows, :] = (key >> 16).astype(jnp.int16)
    lo_scratch[rows, :] = ((key & 0xFFFF) ^ 0x8000).astype(jnp.int16)

    @pl.when(s == n_steps - 1)
    def _select():
        def fold_count(m):
            v = m.astype(jnp.int16)
            r = v.shape[0]
            while r > 16:
                r //= 2
                v = v[:r] + v[r:]
            return jnp.sum(v.astype(jnp.int32))

        def count_hi_ge(thr_i32):
            return fold_count(hi_scratch[...] >= thr_i32.astype(jnp.int16))

        def body_hi(it, t_h):
            cand = t_h | (jnp.int32(1) << (14 - it))
            return jnp.where(count_hi_ge(cand) >= k, cand, t_h)

        t_h = jax.lax.fori_loop(0, 15, body_hi, jnp.int32(0))
        eq_hi = hi_scratch[...] == t_h.astype(jnp.int16)

        cnt_above = fold_count(hi_scratch[...] > t_h.astype(jnp.int16))

        def body_lo(it, t_l):
            cand = t_l | (jnp.int32(1) << (15 - it))
            thr16 = (cand ^ 0x8000).astype(jnp.int16)
            cnt = cnt_above + fold_count(eq_hi & (lo_scratch[...] >= thr16))
            return jnp.where(cnt >= k, cand, t_l)

        t_l = jax.lax.fori_loop(0, 16, body_lo, jnp.int32(0))
        t = (t_h << 16) | t_l

        data = keys_scratch[...]
        gt = data > t
        cnt_gt = jnp.sum(gt.astype(jnp.int32))
        sum_gt = jnp.sum(jnp.where(
            gt, jax.lax.bitcast_convert_type(data, jnp.float32), 0.0))
        t_f = jax.lax.bitcast_convert_type(t, jnp.float32)
        out_ref[0, 0] = (sum_gt + (k - cnt_gt).astype(jnp.float32) * t_f) \
            / jnp.float32(k)


def kernel(logits, labels):
    b, c, h, w = logits.shape
    hw = h * w
    rows = hw // _LANE
    n_chunks = rows // _SUB
    n_steps = b * n_chunks
    k = max(1, int(b * hw * _KEEP_RATIO))

    logits4 = logits.reshape(b, c, rows, _LANE)
    labels4 = labels.reshape(b, 1, rows, _LANE)

    body = functools.partial(_ohem_kernel, n_steps=n_steps, k=k)

    out = pl.pallas_call(
        body,
        grid=(n_steps,),
        in_specs=[
            pl.BlockSpec((1, c, _SUB, _LANE),
                         lambda s: (s // n_chunks, 0, s % n_chunks, 0)),
            pl.BlockSpec((1, 1, _SUB, _LANE),
                         lambda s: (s // n_chunks, 0, s % n_chunks, 0)),
        ],
        out_specs=pl.BlockSpec(memory_space=pltpu.SMEM),
        out_shape=jax.ShapeDtypeStruct((1, 1), jnp.float32),
        scratch_shapes=[
            pltpu.VMEM((n_steps * _SUB, _LANE), jnp.int32),
            pltpu.VMEM((n_steps * _SUB, _LANE), jnp.int16),
            pltpu.VMEM((n_steps * _SUB, _LANE), jnp.int16),
        ],
    )(logits4, labels4)
    return out[0, 0]


# speedup vs baseline: 9.1096x; 1.0247x over previous
import functools

import jax
import jax.numpy as jnp
from jax.experimental import pallas as pl
from jax.experimental.pallas import tpu as pltpu

_C = 19
_SUB = 32
_LANE = 4096
_KEEP_RATIO = 0.25


def _ohem_kernel(logits_ref, labels_ref, out_ref, keys_scratch, hi_scratch,
                 lo_scratch, *, n_steps, k):
    s = pl.program_id(0)
    x = logits_ref[0]
    lab = labels_ref[0, 0]

    mx = jnp.max(x, axis=0)
    lse = mx + jnp.log(jnp.sum(jnp.exp(x - mx[None]), axis=0))
    cls = jax.lax.broadcasted_iota(jnp.int32, x.shape, 0)
    sel = jnp.sum(jnp.where(cls == lab[None], x, 0.0), axis=0)
    loss = lse - sel
    key = jax.lax.bitcast_convert_type(loss, jnp.int32)
    rows = pl.ds(s * _SUB, _SUB)
    keys_scratch[rows, :] = key
    hi_scratch[rows, :] = (key >> 16).astype(jnp.int16)
    lo_scratch[rows, :] = ((key & 0xFFFF) ^ 0x8000).astype(jnp.int16)

    @pl.when(s == n_steps - 1)
    def _select():
        def fold_count(m):
            v = m.astype(jnp.int16)
            r = v.shape[0]
            while r > 16:
                r //= 2
                v = v[:r] + v[r:]
            return jnp.sum(v.astype(jnp.int32))

        def count_hi_ge(thr_i32):
            return fold_count(hi_scratch[...] >= thr_i32.astype(jnp.int16))

        def body_hi(it, t_h):
            cand = t_h | (jnp.int32(1) << (14 - it))
            return jnp.where(count_hi_ge(cand) >= k, cand, t_h)

        t_h = jax.lax.fori_loop(0, 15, body_hi, jnp.int32(0))
        eq_hi = hi_scratch[...] == t_h.astype(jnp.int16)

        cnt_above = fold_count(hi_scratch[...] > t_h.astype(jnp.int16))

        def body_lo(it, t_l):
            cand = t_l | (jnp.int32(1) << (15 - it))
            thr16 = (cand ^ 0x8000).astype(jnp.int16)
            cnt = cnt_above + fold_count(eq_hi & (lo_scratch[...] >= thr16))
            return jnp.where(cnt >= k, cand, t_l)

        t_l = jax.lax.fori_loop(0, 16, body_lo, jnp.int32(0))
        t = (t_h << 16) | t_l

        data = keys_scratch[...]
        gt = data > t
        cnt_gt = jnp.sum(gt.astype(jnp.int32))
        sum_gt = jnp.sum(jnp.where(
            gt, jax.lax.bitcast_convert_type(data, jnp.float32), 0.0))
        t_f = jax.lax.bitcast_convert_type(t, jnp.float32)
        out_ref[0, 0] = (sum_gt + (k - cnt_gt).astype(jnp.float32) * t_f) \
            / jnp.float32(k)


def kernel(logits, labels):
    b, c, h, w = logits.shape
    hw = h * w
    rows = hw // _LANE
    n_chunks = rows // _SUB
    n_steps = b * n_chunks
    k = max(1, int(b * hw * _KEEP_RATIO))

    logits4 = logits.reshape(b, c, rows, _LANE)
    labels4 = labels.reshape(b, 1, rows, _LANE)

    body = functools.partial(_ohem_kernel, n_steps=n_steps, k=k)

    out = pl.pallas_call(
        body,
        grid=(n_steps,),
        in_specs=[
            pl.BlockSpec((1, c, _SUB, _LANE),
                         lambda s: (s // n_chunks, 0, s % n_chunks, 0)),
            pl.BlockSpec((1, 1, _SUB, _LANE),
                         lambda s: (s // n_chunks, 0, s % n_chunks, 0)),
        ],
        out_specs=pl.BlockSpec(memory_space=pltpu.SMEM),
        out_shape=jax.ShapeDtypeStruct((1, 1), jnp.float32),
        scratch_shapes=[
            pltpu.VMEM((n_steps * _SUB, _LANE), jnp.int32),
            pltpu.VMEM((n_steps * _SUB, _LANE), jnp.int16),
            pltpu.VMEM((n_steps * _SUB, _LANE), jnp.int16),
        ],
    )(logits4, labels4)
    return out[0, 0]
